# baseline (speedup 1.0000x reference)
import functools

import jax
import jax.numpy as jnp
from jax import lax
from jax.experimental import pallas as pl
from jax.experimental.pallas import tpu as pltpu
from jax.experimental.pallas import tpu_sc as plsc

EMB = 128
ABLK = 2000
NC = 2
NS = 16
CHUNK = 128
BI = 20
ACC_ROWS = 10112


def _swish(x):
    return x * jax.nn.sigmoid(x)


def _embed_body(z_ref, emb_ref, wm_ref, bm_ref, h_ref, m_ref):
    z = z_ref[...]
    io = lax.broadcasted_iota(jnp.int32, (z.shape[0], EMB), 1)
    oh = (io == z).astype(jnp.float32)
    h = jnp.dot(oh, emb_ref[...], preferred_element_type=jnp.float32, precision=jax.lax.Precision.HIGHEST)
    h_ref[...] = h
    m_ref[...] = _swish(
        jnp.dot(h, wm_ref[...], preferred_element_type=jnp.float32, precision=jax.lax.Precision.HIGHEST) + bm_ref[...])


def _update_body(p_ref, h_ref, wu_ref, bu_ref, wn_ref, bn_ref, hn_ref, mn_ref):
    agg = p_ref[0] + p_ref[1]
    u = _swish(
        jnp.dot(agg, wu_ref[...], preferred_element_type=jnp.float32, precision=jax.lax.Precision.HIGHEST) + bu_ref[...])
    hn = h_ref[...] + u
    hn_ref[...] = hn
    mn_ref[...] = _swish(
        jnp.dot(hn, wn_ref[...], preferred_element_type=jnp.float32, precision=jax.lax.Precision.HIGHEST) + bn_ref[...])


def _dens_body(ho_ref, wsp_ref, bsp_ref, r_ref, ct_ref, al_ref, o_ref):
    sp = jnp.dot(ho_ref[...], wsp_ref[...],
                 preferred_element_type=jnp.float32, precision=jax.lax.Precision.HIGHEST) + bsp_ref[...]
    rb = r_ref[...]
    ct = ct_ref[...]
    a = al_ref[...]
    rdc = jnp.dot(rb, ct, preferred_element_type=jnp.float32, precision=jax.lax.Precision.HIGHEST)
    rsq = jnp.sum(rb * rb, axis=1, keepdims=True)
    csq = jnp.sum(ct * ct, axis=0, keepdims=True)
    r2 = rsq + csq - 2.0 * rdc
    acc = jnp.zeros_like(r2)
    for s in range(4):
        acc = acc + sp[:, s:s + 1] * jnp.exp(r2 * (-jnp.abs(a[0:1, s:s + 1])))
    for p in range(2):
        cpp = sp[:, 4 + 3 * p:7 + 3 * p]
        pdc = jnp.dot(cpp, ct[0:3, :], preferred_element_type=jnp.float32, precision=jax.lax.Precision.HIGHEST)
        prd = jnp.sum(cpp * rb[:, 0:3], axis=1, keepdims=True)
        acc = acc + (pdc - prd) * jnp.exp(r2 * (-jnp.abs(a[0:1, 4 + p:5 + p])))
    col = jnp.sum(acc, axis=0, keepdims=True)

    @pl.when(pl.program_id(0) == 0)
    def _():
        o_ref[...] = jnp.zeros_like(o_ref)

    o_ref[...] += col


def _segsum_body(m_hbm, ej_hbm, ei_hbm, z_hbm, out_hbm,
                 ejb0, ejb1, eib0, eib1, buf0, buf1, acc_sh, gsem, ssem, isem):
    cid = lax.axis_index("c")
    sid = lax.axis_index("s")
    wid = sid * NC + cid
    rpt = ACC_ROWS // NS
    nb = ej_hbm.shape[1]
    bufs = (buf0, buf1)
    ejbs = (ejb0, ejb1)
    eibs = (eib0, eib1)
    dummy = m_hbm.at[pl.ds(0, CHUNK)]

    pltpu.sync_copy(z_hbm, acc_sh.at[pl.ds(sid * rpt, rpt)])
    pltpu.async_copy(ej_hbm.at[wid, 0], ejb0, isem)
    pltpu.async_copy(ei_hbm.at[wid, 0], eib0, isem)
    pltpu.make_async_copy(ej_hbm.at[wid, 0], ejb0, isem).wait()
    pltpu.make_async_copy(ei_hbm.at[wid, 0], eib0, isem).wait()
    plsc.subcore_barrier()

    for k in range(nb):
        ejb, eib = ejbs[k % 2], eibs[k % 2]
        ejb_n, eib_n = ejbs[1 - k % 2], eibs[1 - k % 2]

        def chunk_step(g, carry, ejb=ejb, eib=eib):
            pltpu.async_copy(m_hbm.at[ejb.at[g]], buf0, gsem).wait()
            pltpu.sync_copy(buf0, acc_sh.at[eib.at[g]], add=True)
            return carry

        lax.fori_loop(0, BI, chunk_step, 0)
        if k + 1 < nb:
            pltpu.async_copy(ej_hbm.at[wid, k + 1], ejb_n, isem)
            pltpu.async_copy(ei_hbm.at[wid, k + 1], eib_n, isem)
            pltpu.make_async_copy(ej_hbm.at[wid, k + 1], ejb_n, isem).wait()
            pltpu.make_async_copy(ei_hbm.at[wid, k + 1], eib_n, isem).wait()
    plsc.subcore_barrier()
    pltpu.sync_copy(acc_sh.at[pl.ds(sid * rpt, rpt)],
                    out_hbm.at[cid, pl.ds(sid * rpt, rpt)])


@functools.cache
def _make_segsum(nb):
    return functools.partial(
        pl.kernel,
        mesh=plsc.VectorSubcoreMesh(core_axis_name="c", subcore_axis_name="s"),
        out_type=jax.ShapeDtypeStruct((NC, ACC_ROWS, EMB), jnp.float32),
        scratch_types=[
            pltpu.VMEM((BI, CHUNK), jnp.int32),
            pltpu.VMEM((BI, CHUNK), jnp.int32),
            pltpu.VMEM((BI, CHUNK), jnp.int32),
            pltpu.VMEM((BI, CHUNK), jnp.int32),
            pltpu.VMEM((CHUNK, EMB), jnp.float32),
            pltpu.VMEM((CHUNK, EMB), jnp.float32),
            pltpu.VMEM_SHARED((ACC_ROWS, EMB), jnp.float32),
            pltpu.SemaphoreType.DMA,
            pltpu.SemaphoreType.DMA,
            pltpu.SemaphoreType.DMA,
        ],
    )(_segsum_body)


def _segsum(m, ej, ei, zeros_src):
    return _make_segsum(ej.shape[1])(m, ej, ei, zeros_src)


def _row_specs(n):
    return [pl.BlockSpec((ABLK, EMB), lambda i: (i, 0)) for _ in range(n)]


def _full(shape):
    return pl.BlockSpec(shape, lambda i: tuple(0 for _ in shape))


def kernel(Z, N, R, edge_id_i, edge_id_j, coords, emb_table, Wm, bm, Wu, bu,
           Wo1, bo1, Ws, bs, Wp, bp, alpha_s, alpha_p):
    A = Z.shape[0]
    E = edge_id_i.shape[0]
    G = coords.shape[0]
    grid = (A // ABLK,)

    emb_pad = jnp.zeros((EMB, EMB), jnp.float32).at[:emb_table.shape[0]].set(emb_table)
    estep = NC * NS * CHUNK * BI
    e_pad = ((E + estep - 1) // estep) * estep
    ej = edge_id_j.astype(jnp.int32)
    ei = edge_id_i.astype(jnp.int32)
    if e_pad > E:
        ej = jnp.concatenate([ej, jnp.zeros((e_pad - E,), jnp.int32)])
        ei = jnp.concatenate([ei, jnp.full((e_pad - E,), A, jnp.int32)])
    ej = ej.reshape(NC * NS, -1, BI, CHUNK)
    ei = ei.reshape(NC * NS, -1, BI, CHUNK)
    zeros_src = jnp.zeros((ACC_ROWS // NS, EMB), jnp.float32)

    h, m = pl.pallas_call(
        _embed_body,
        grid=grid,
        in_specs=[pl.BlockSpec((ABLK, 1), lambda i: (i, 0)),
                  _full((EMB, EMB)), _full((EMB, EMB)), _full((1, EMB))],
        out_specs=_row_specs(2),
        out_shape=[jax.ShapeDtypeStruct((A, EMB), jnp.float32)] * 2,
    )(Z.reshape(A, 1).astype(jnp.int32), emb_pad, Wm[0], bm[0].reshape(1, EMB))

    NUM_BLOCKS = Wm.shape[0]
    for t in range(NUM_BLOCKS):
        parts = _segsum(m, ej, ei, zeros_src)
        if t < NUM_BLOCKS - 1:
            wn, bn = Wm[t + 1], bm[t + 1]
        else:
            wn, bn = Wo1, bo1
        h, m = pl.pallas_call(
            _update_body,
            grid=grid,
            in_specs=[pl.BlockSpec((NC, ABLK, EMB), lambda i: (0, i, 0)),
                      pl.BlockSpec((ABLK, EMB), lambda i: (i, 0)),
                      _full((EMB, EMB)), _full((1, EMB)),
                      _full((EMB, EMB)), _full((1, EMB))],
            out_specs=_row_specs(2),
            out_shape=[jax.ShapeDtypeStruct((A, EMB), jnp.float32)] * 2,
        )(parts, h, Wu[t], bu[t].reshape(1, EMB), wn, bn.reshape(1, EMB))

    wsp = jnp.zeros((EMB, 16), jnp.float32).at[:, 0:4].set(Ws).at[:, 4:10].set(Wp)
    bsp = jnp.zeros((1, 16), jnp.float32).at[0, 0:4].set(bs).at[0, 4:10].set(bp)
    rpad = jnp.zeros((A, 8), jnp.float32).at[:, 0:3].set(R)
    ct = jnp.zeros((8, G), jnp.float32).at[0:3, :].set(coords.T)
    al = jnp.zeros((1, 8), jnp.float32).at[0, 0:4].set(alpha_s).at[0, 4:6].set(alpha_p)
    dens = pl.pallas_call(
        _dens_body,
        grid=grid,
        in_specs=[pl.BlockSpec((ABLK, EMB), lambda i: (i, 0)),
                  _full((EMB, 16)), _full((1, 16)),
                  pl.BlockSpec((ABLK, 8), lambda i: (i, 0)),
                  _full((8, G)), _full((1, 8))],
        out_specs=_full((1, G)),
        out_shape=jax.ShapeDtypeStruct((1, G), jnp.float32),
    )(m, wsp, bsp, rpad, ct, al)
    return dens.reshape(G)


# speedup vs baseline: 1.1978x; 1.1978x over previous
import functools

import jax
import jax.numpy as jnp
from jax import lax
from jax.experimental import pallas as pl
from jax.experimental.pallas import tpu as pltpu
from jax.experimental.pallas import tpu_sc as plsc

EMB = 128
ABLK = 2000
NC = 2
NS = 16
CHUNK = 128
ACC_ROWS = 10112


def _swish(x):
    return x * jax.nn.sigmoid(x)


def _embed_body(z_ref, emb_ref, wm_ref, bm_ref, h_ref, m_ref):
    z = z_ref[...]
    io = lax.broadcasted_iota(jnp.int32, (z.shape[0], EMB), 1)
    oh = (io == z).astype(jnp.float32)
    h = jnp.dot(oh, emb_ref[...], preferred_element_type=jnp.float32, precision=jax.lax.Precision.HIGHEST)
    h_ref[...] = h
    m_ref[...] = _swish(
        jnp.dot(h, wm_ref[...], preferred_element_type=jnp.float32, precision=jax.lax.Precision.HIGHEST) + bm_ref[...])


def _update_body(p_ref, h_ref, wu_ref, bu_ref, wn_ref, bn_ref, hn_ref, mn_ref):
    agg = p_ref[0] + p_ref[1]
    u = _swish(
        jnp.dot(agg, wu_ref[...], preferred_element_type=jnp.float32, precision=jax.lax.Precision.HIGHEST) + bu_ref[...])
    hn = h_ref[...] + u
    hn_ref[...] = hn
    mn_ref[...] = _swish(
        jnp.dot(hn, wn_ref[...], preferred_element_type=jnp.float32, precision=jax.lax.Precision.HIGHEST) + bn_ref[...])


def _dens_body(ho_ref, wsp_ref, bsp_ref, r_ref, ct_ref, al_ref, o_ref):
    sp = jnp.dot(ho_ref[...], wsp_ref[...],
                 preferred_element_type=jnp.float32, precision=jax.lax.Precision.HIGHEST) + bsp_ref[...]
    rb = r_ref[...]
    ct = ct_ref[...]
    a = al_ref[...]
    rdc = jnp.dot(rb, ct, preferred_element_type=jnp.float32, precision=jax.lax.Precision.HIGHEST)
    rsq = jnp.sum(rb * rb, axis=1, keepdims=True)
    csq = jnp.sum(ct * ct, axis=0, keepdims=True)
    r2 = rsq + csq - 2.0 * rdc
    acc = jnp.zeros_like(r2)
    for s in range(4):
        acc = acc + sp[:, s:s + 1] * jnp.exp(r2 * (-jnp.abs(a[0:1, s:s + 1])))
    for p in range(2):
        cpp = sp[:, 4 + 3 * p:7 + 3 * p]
        pdc = jnp.dot(cpp, ct[0:3, :], preferred_element_type=jnp.float32, precision=jax.lax.Precision.HIGHEST)
        prd = jnp.sum(cpp * rb[:, 0:3], axis=1, keepdims=True)
        acc = acc + (pdc - prd) * jnp.exp(r2 * (-jnp.abs(a[0:1, 4 + p:5 + p])))
    col = jnp.sum(acc, axis=0, keepdims=True)

    @pl.when(pl.program_id(0) == 0)
    def _():
        o_ref[...] = jnp.zeros_like(o_ref)

    o_ref[...] += col


def _segsum_body(m_hbm, ej_hbm, ei_hbm, z_hbm, out_hbm,
                 ej0, ej1, ei0, ei1, buf0, buf1, acc_sh, gsem, isem):
    cid = lax.axis_index("c")
    sid = lax.axis_index("s")
    wid = sid * NC + cid
    rpt = ACC_ROWS // NS
    ch = ej_hbm.shape[0] // (NC * NS * CHUNK)
    bufs = (buf0, buf1)
    ejs = (ej0, ej1)
    eis = (ei0, ei1)
    dummy = m_hbm.at[pl.ds(0, CHUNK)]

    pltpu.sync_copy(z_hbm, acc_sh.at[pl.ds(sid * rpt, rpt)])
    base0 = wid * ch * CHUNK
    pltpu.sync_copy(ej_hbm.at[pl.ds(base0, CHUNK)], ej0)
    pltpu.sync_copy(ei_hbm.at[pl.ds(base0, CHUNK)], ei0)
    pltpu.async_copy(ej_hbm.at[pl.ds(base0 + CHUNK, CHUNK)], ej1, isem)
    pltpu.async_copy(ei_hbm.at[pl.ds(base0 + CHUNK, CHUNK)], ei1, isem)
    pltpu.async_copy(m_hbm.at[ej0], buf0, gsem)
    plsc.subcore_barrier()

    def pair(i, carry):
        for b in (0, 1):
            g = 2 * i + b
            mine, other = bufs[b], bufs[1 - b]
            ejn, ein = ejs[1 - b], eis[1 - b]
            pltpu.make_async_copy(dummy, mine, gsem).wait()

            @pl.when(g + 1 < ch)
            def _():
                pltpu.make_async_copy(ej_hbm.at[pl.ds(0, CHUNK)], ejn, isem).wait()
                pltpu.make_async_copy(ei_hbm.at[pl.ds(0, CHUNK)], ein, isem).wait()
                pltpu.async_copy(m_hbm.at[ejn], other, gsem)

            pltpu.sync_copy(mine, acc_sh.at[eis[b]], add=True)

            @pl.when(g + 2 < ch)
            def _():
                base = (wid * ch + g + 2) * CHUNK
                pltpu.async_copy(ej_hbm.at[pl.ds(base, CHUNK)], ejs[b], isem)
                pltpu.async_copy(ei_hbm.at[pl.ds(base, CHUNK)], eis[b], isem)
        return carry

    lax.fori_loop(0, ch // 2, pair, 0)
    plsc.subcore_barrier()
    pltpu.sync_copy(acc_sh.at[pl.ds(sid * rpt, rpt)],
                    out_hbm.at[cid, pl.ds(sid * rpt, rpt)])


@functools.cache
def _make_segsum():
    return functools.partial(
        pl.kernel,
        mesh=plsc.VectorSubcoreMesh(core_axis_name="c", subcore_axis_name="s"),
        out_type=jax.ShapeDtypeStruct((NC, ACC_ROWS, EMB), jnp.float32),
        scratch_types=[
            pltpu.VMEM((CHUNK,), jnp.int32),
            pltpu.VMEM((CHUNK,), jnp.int32),
            pltpu.VMEM((CHUNK,), jnp.int32),
            pltpu.VMEM((CHUNK,), jnp.int32),
            pltpu.VMEM((CHUNK, EMB), jnp.float32),
            pltpu.VMEM((CHUNK, EMB), jnp.float32),
            pltpu.VMEM_SHARED((ACC_ROWS, EMB), jnp.float32),
            pltpu.SemaphoreType.DMA,
            pltpu.SemaphoreType.DMA,
        ],
    )(_segsum_body)


def _segsum(m, ej, ei, zeros_src):
    return _make_segsum()(m, ej, ei, zeros_src)


def _row_specs(n):
    return [pl.BlockSpec((ABLK, EMB), lambda i: (i, 0)) for _ in range(n)]


def _full(shape):
    return pl.BlockSpec(shape, lambda i: tuple(0 for _ in shape))


def kernel(Z, N, R, edge_id_i, edge_id_j, coords, emb_table, Wm, bm, Wu, bu,
           Wo1, bo1, Ws, bs, Wp, bp, alpha_s, alpha_p):
    A = Z.shape[0]
    E = edge_id_i.shape[0]
    G = coords.shape[0]
    grid = (A // ABLK,)

    emb_pad = jnp.zeros((EMB, EMB), jnp.float32).at[:emb_table.shape[0]].set(emb_table)
    estep = NC * NS * CHUNK * 2
    e_pad = ((E + estep - 1) // estep) * estep
    ej = edge_id_j.astype(jnp.int32)
    ei = edge_id_i.astype(jnp.int32)
    if e_pad > E:
        ej = jnp.concatenate([ej, jnp.zeros((e_pad - E,), jnp.int32)])
        ei = jnp.concatenate([ei, jnp.full((e_pad - E,), A, jnp.int32)])
    zeros_src = jnp.zeros((ACC_ROWS // NS, EMB), jnp.float32)

    h, m = pl.pallas_call(
        _embed_body,
        grid=grid,
        in_specs=[pl.BlockSpec((ABLK, 1), lambda i: (i, 0)),
                  _full((EMB, EMB)), _full((EMB, EMB)), _full((1, EMB))],
        out_specs=_row_specs(2),
        out_shape=[jax.ShapeDtypeStruct((A, EMB), jnp.float32)] * 2,
    )(Z.reshape(A, 1).astype(jnp.int32), emb_pad, Wm[0], bm[0].reshape(1, EMB))

    NUM_BLOCKS = Wm.shape[0]
    for t in range(NUM_BLOCKS):
        parts = _segsum(m, ej, ei, zeros_src)
        if t < NUM_BLOCKS - 1:
            wn, bn = Wm[t + 1], bm[t + 1]
        else:
            wn, bn = Wo1, bo1
        h, m = pl.pallas_call(
            _update_body,
            grid=grid,
            in_specs=[pl.BlockSpec((NC, ABLK, EMB), lambda i: (0, i, 0)),
                      pl.BlockSpec((ABLK, EMB), lambda i: (i, 0)),
                      _full((EMB, EMB)), _full((1, EMB)),
                      _full((EMB, EMB)), _full((1, EMB))],
            out_specs=_row_specs(2),
            out_shape=[jax.ShapeDtypeStruct((A, EMB), jnp.float32)] * 2,
        )(parts, h, Wu[t], bu[t].reshape(1, EMB), wn, bn.reshape(1, EMB))

    wsp = jnp.zeros((EMB, 16), jnp.float32).at[:, 0:4].set(Ws).at[:, 4:10].set(Wp)
    bsp = jnp.zeros((1, 16), jnp.float32).at[0, 0:4].set(bs).at[0, 4:10].set(bp)
    rpad = jnp.zeros((A, 8), jnp.float32).at[:, 0:3].set(R)
    ct = jnp.zeros((8, G), jnp.float32).at[0:3, :].set(coords.T)
    al = jnp.zeros((1, 8), jnp.float32).at[0, 0:4].set(alpha_s).at[0, 4:6].set(alpha_p)
    dens = pl.pallas_call(
        _dens_body,
        grid=grid,
        in_specs=[pl.BlockSpec((ABLK, EMB), lambda i: (i, 0)),
                  _full((EMB, 16)), _full((1, 16)),
                  pl.BlockSpec((ABLK, 8), lambda i: (i, 0)),
                  _full((8, G)), _full((1, 8))],
        out_specs=_full((1, G)),
        out_shape=jax.ShapeDtypeStruct((1, G), jnp.float32),
    )(m, wsp, bsp, rpad, ct, al)
    return dens.reshape(G)


# speedup vs baseline: 1.1987x; 1.0007x over previous
import functools

import jax
import jax.numpy as jnp
from jax import lax
from jax.experimental import pallas as pl
from jax.experimental.pallas import tpu as pltpu
from jax.experimental.pallas import tpu_sc as plsc

EMB = 128
ABLK = 2000
NC = 2
NS = 16
CHUNK = 128
ACC_ROWS = 10112


def _swish(x):
    return x * jax.nn.sigmoid(x)


def _embed_body(z_ref, emb_ref, wm_ref, bm_ref, h_ref, m_ref):
    z = z_ref[...]
    io = lax.broadcasted_iota(jnp.int32, (z.shape[0], EMB), 1)
    oh = (io == z).astype(jnp.float32)
    h = jnp.dot(oh, emb_ref[...], preferred_element_type=jnp.float32, precision=jax.lax.Precision.HIGHEST)
    h_ref[...] = h
    m_ref[...] = _swish(
        jnp.dot(h, wm_ref[...], preferred_element_type=jnp.float32, precision=jax.lax.Precision.HIGHEST) + bm_ref[...])


def _update_body(p_ref, h_ref, wu_ref, bu_ref, wn_ref, bn_ref, hn_ref, mn_ref):
    agg = p_ref[0] + p_ref[1]
    u = _swish(
        jnp.dot(agg, wu_ref[...], preferred_element_type=jnp.float32, precision=jax.lax.Precision.HIGHEST) + bu_ref[...])
    hn = h_ref[...] + u
    hn_ref[...] = hn
    mn_ref[...] = _swish(
        jnp.dot(hn, wn_ref[...], preferred_element_type=jnp.float32, precision=jax.lax.Precision.HIGHEST) + bn_ref[...])


def _dens_body(ho_ref, wsp_ref, bsp_ref, r_ref, ct_ref, al_ref, o_ref):
    sp = jnp.dot(ho_ref[...], wsp_ref[...],
                 preferred_element_type=jnp.float32, precision=jax.lax.Precision.HIGHEST) + bsp_ref[...]
    rb = r_ref[...]
    ct = ct_ref[...]
    a = al_ref[...]
    rdc = jnp.dot(rb, ct, preferred_element_type=jnp.float32, precision=jax.lax.Precision.HIGHEST)
    rsq = jnp.sum(rb * rb, axis=1, keepdims=True)
    csq = jnp.sum(ct * ct, axis=0, keepdims=True)
    r2 = rsq + csq - 2.0 * rdc
    acc = jnp.zeros_like(r2)
    for s in range(4):
        acc = acc + sp[:, s:s + 1] * jnp.exp(r2 * (-jnp.abs(a[0:1, s:s + 1])))
    for p in range(2):
        cpp = sp[:, 4 + 3 * p:7 + 3 * p]
        pdc = jnp.dot(cpp, ct[0:3, :], preferred_element_type=jnp.float32, precision=jax.lax.Precision.HIGHEST)
        prd = jnp.sum(cpp * rb[:, 0:3], axis=1, keepdims=True)
        acc = acc + (pdc - prd) * jnp.exp(r2 * (-jnp.abs(a[0:1, 4 + p:5 + p])))
    col = jnp.sum(acc, axis=0, keepdims=True)

    @pl.when(pl.program_id(0) == 0)
    def _():
        o_ref[...] = jnp.zeros_like(o_ref)

    o_ref[...] += col


def _segsum_body(m_hbm, ej_hbm, ei_hbm, z_hbm, out_hbm,
                 ej1d, eib, buf0, buf1, acc_sh, gsem):
    cid = lax.axis_index("c")
    sid = lax.axis_index("s")
    wid = sid * NC + cid
    rpt = ACC_ROWS // NS
    ch = ej_hbm.shape[0] // (NC * NS * CHUNK)
    bufs = (buf0, buf1)
    dummy = m_hbm.at[pl.ds(0, CHUNK)]

    pltpu.sync_copy(z_hbm, acc_sh.at[pl.ds(sid * rpt, rpt)])
    pltpu.sync_copy(ej_hbm.at[pl.ds(wid * ch * CHUNK, ch * CHUNK)], ej1d)
    pltpu.async_copy(m_hbm.at[ej1d.at[pl.ds(0, CHUNK)]], buf0, gsem)
    plsc.subcore_barrier()

    def pair(i, carry):
        for b in (0, 1):
            g = 2 * i + b
            mine, other = bufs[b], bufs[1 - b]
            pltpu.make_async_copy(dummy, mine, gsem).wait()

            @pl.when(g + 1 < ch)
            def _():
                pltpu.async_copy(
                    m_hbm.at[ej1d.at[pl.ds((g + 1) * CHUNK, CHUNK)]],
                    other, gsem)

            pltpu.sync_copy(
                ei_hbm.at[pl.ds((wid * ch + g) * CHUNK, CHUNK)], eib)
            pltpu.sync_copy(mine, acc_sh.at[eib], add=True)
        return carry

    lax.fori_loop(0, ch // 2, pair, 0)
    plsc.subcore_barrier()
    pltpu.sync_copy(acc_sh.at[pl.ds(sid * rpt, rpt)],
                    out_hbm.at[cid, pl.ds(sid * rpt, rpt)])


@functools.cache
def _make_segsum(ch):
    return functools.partial(
        pl.kernel,
        mesh=plsc.VectorSubcoreMesh(core_axis_name="c", subcore_axis_name="s"),
        out_type=jax.ShapeDtypeStruct((NC, ACC_ROWS, EMB), jnp.float32),
        scratch_types=[
            pltpu.VMEM((ch * CHUNK,), jnp.int32),
            pltpu.VMEM((CHUNK,), jnp.int32),
            pltpu.VMEM((CHUNK, EMB), jnp.float32),
            pltpu.VMEM((CHUNK, EMB), jnp.float32),
            pltpu.VMEM_SHARED((ACC_ROWS, EMB), jnp.float32),
            pltpu.SemaphoreType.DMA,
        ],
    )(_segsum_body)


def _segsum(m, ej, ei, zeros_src):
    ch = ej.shape[0] // (NC * NS * CHUNK)
    return _make_segsum(ch)(m, ej, ei, zeros_src)


def _row_specs(n):
    return [pl.BlockSpec((ABLK, EMB), lambda i: (i, 0)) for _ in range(n)]


def _full(shape):
    return pl.BlockSpec(shape, lambda i: tuple(0 for _ in shape))


def kernel(Z, N, R, edge_id_i, edge_id_j, coords, emb_table, Wm, bm, Wu, bu,
           Wo1, bo1, Ws, bs, Wp, bp, alpha_s, alpha_p):
    A = Z.shape[0]
    E = edge_id_i.shape[0]
    G = coords.shape[0]
    grid = (A // ABLK,)

    emb_pad = jnp.zeros((EMB, EMB), jnp.float32).at[:emb_table.shape[0]].set(emb_table)
    estep = NC * NS * CHUNK * 2
    e_pad = ((E + estep - 1) // estep) * estep
    ej = edge_id_j.astype(jnp.int32)
    ei = edge_id_i.astype(jnp.int32)
    if e_pad > E:
        ej = jnp.concatenate([ej, jnp.zeros((e_pad - E,), jnp.int32)])
        ei = jnp.concatenate([ei, jnp.full((e_pad - E,), A, jnp.int32)])
    zeros_src = jnp.zeros((ACC_ROWS // NS, EMB), jnp.float32)

    h, m = pl.pallas_call(
        _embed_body,
        grid=grid,
        in_specs=[pl.BlockSpec((ABLK, 1), lambda i: (i, 0)),
                  _full((EMB, EMB)), _full((EMB, EMB)), _full((1, EMB))],
        out_specs=_row_specs(2),
        out_shape=[jax.ShapeDtypeStruct((A, EMB), jnp.float32)] * 2,
    )(Z.reshape(A, 1).astype(jnp.int32), emb_pad, Wm[0], bm[0].reshape(1, EMB))

    NUM_BLOCKS = Wm.shape[0]
    for t in range(NUM_BLOCKS):
        parts = _segsum(m, ej, ei, zeros_src)
        if t < NUM_BLOCKS - 1:
            wn, bn = Wm[t + 1], bm[t + 1]
        else:
            wn, bn = Wo1, bo1
        h, m = pl.pallas_call(
            _update_body,
            grid=grid,
            in_specs=[pl.BlockSpec((NC, ABLK, EMB), lambda i: (0, i, 0)),
                      pl.BlockSpec((ABLK, EMB), lambda i: (i, 0)),
                      _full((EMB, EMB)), _full((1, EMB)),
                      _full((EMB, EMB)), _full((1, EMB))],
            out_specs=_row_specs(2),
            out_shape=[jax.ShapeDtypeStruct((A, EMB), jnp.float32)] * 2,
        )(parts, h, Wu[t], bu[t].reshape(1, EMB), wn, bn.reshape(1, EMB))

    wsp = jnp.zeros((EMB, 16), jnp.float32).at[:, 0:4].set(Ws).at[:, 4:10].set(Wp)
    bsp = jnp.zeros((1, 16), jnp.float32).at[0, 0:4].set(bs).at[0, 4:10].set(bp)
    rpad = jnp.zeros((A, 8), jnp.float32).at[:, 0:3].set(R)
    ct = jnp.zeros((8, G), jnp.float32).at[0:3, :].set(coords.T)
    al = jnp.zeros((1, 8), jnp.float32).at[0, 0:4].set(alpha_s).at[0, 4:6].set(alpha_p)
    dens = pl.pallas_call(
        _dens_body,
        grid=grid,
        in_specs=[pl.BlockSpec((ABLK, EMB), lambda i: (i, 0)),
                  _full((EMB, 16)), _full((1, 16)),
                  pl.BlockSpec((ABLK, 8), lambda i: (i, 0)),
                  _full((8, G)), _full((1, 8))],
        out_specs=_full((1, G)),
        out_shape=jax.ShapeDtypeStruct((1, G), jnp.float32),
    )(m, wsp, bsp, rpad, ct, al)
    return dens.reshape(G)


# speedup vs baseline: 1.7919x; 1.4949x over previous
import functools

import jax
import jax.numpy as jnp
from jax import lax
from jax.experimental import pallas as pl
from jax.experimental.pallas import tpu as pltpu
from jax.experimental.pallas import tpu_sc as plsc

EMB = 128
ABLK = 2000
NC = 2
NS = 16
CHUNK = 128
ACC_ROWS = 10112


def _swish(x):
    return x * jax.nn.sigmoid(x)


def _embed_body(z_ref, emb_ref, wm_ref, bm_ref, h_ref, m_ref):
    z = z_ref[...]
    io = lax.broadcasted_iota(jnp.int32, (z.shape[0], EMB), 1)
    oh = (io == z).astype(jnp.float32)
    h = jnp.dot(oh, emb_ref[...], preferred_element_type=jnp.float32, precision=jax.lax.Precision.HIGHEST)
    h_ref[...] = h
    m_ref[...] = _swish(
        jnp.dot(h, wm_ref[...], preferred_element_type=jnp.float32, precision=jax.lax.Precision.HIGHEST) + bm_ref[...])


def _update_body(p_ref, h_ref, wu_ref, bu_ref, wn_ref, bn_ref, hn_ref, mn_ref):
    agg = p_ref[0] + p_ref[1]
    u = _swish(
        jnp.dot(agg, wu_ref[...], preferred_element_type=jnp.float32, precision=jax.lax.Precision.HIGHEST) + bu_ref[...])
    hn = h_ref[...] + u
    hn_ref[...] = hn
    mn_ref[...] = _swish(
        jnp.dot(hn, wn_ref[...], preferred_element_type=jnp.float32, precision=jax.lax.Precision.HIGHEST) + bn_ref[...])


def _dens_body(ho_ref, wsp_ref, bsp_ref, r_ref, ct_ref, al_ref, o_ref):
    sp = jnp.dot(ho_ref[...], wsp_ref[...],
                 preferred_element_type=jnp.float32, precision=jax.lax.Precision.HIGHEST) + bsp_ref[...]
    rb = r_ref[...]
    ct = ct_ref[...]
    a = al_ref[...]
    rdc = jnp.dot(rb, ct, preferred_element_type=jnp.float32, precision=jax.lax.Precision.HIGHEST)
    rsq = jnp.sum(rb * rb, axis=1, keepdims=True)
    csq = jnp.sum(ct * ct, axis=0, keepdims=True)
    r2 = rsq + csq - 2.0 * rdc
    acc = jnp.zeros_like(r2)
    for s in range(4):
        acc = acc + sp[:, s:s + 1] * jnp.exp(r2 * (-jnp.abs(a[0:1, s:s + 1])))
    for p in range(2):
        cpp = sp[:, 4 + 3 * p:7 + 3 * p]
        pdc = jnp.dot(cpp, ct[0:3, :], preferred_element_type=jnp.float32, precision=jax.lax.Precision.HIGHEST)
        prd = jnp.sum(cpp * rb[:, 0:3], axis=1, keepdims=True)
        acc = acc + (pdc - prd) * jnp.exp(r2 * (-jnp.abs(a[0:1, 4 + p:5 + p])))
    col = jnp.sum(acc, axis=0, keepdims=True)

    @pl.when(pl.program_id(0) == 0)
    def _():
        o_ref[...] = jnp.zeros_like(o_ref)

    o_ref[...] += col


CH0 = 79
CH1 = 78


def _segsum_body(m_hbm, ej_hbm, ei_hbm, z_hbm, out_hbm,
                 idxj_v, idxi_v, rows_v, acc_sh, gsem):
    cid = lax.axis_index("c")
    sid = lax.axis_index("s")
    rpt = ACC_ROWS // NS
    pltpu.sync_copy(z_hbm, acc_sh.at[pl.ds(sid * rpt, rpt)])
    plsc.subcore_barrier()
    my_ch = jnp.where(cid == 0, CH0, CH1)
    base0 = (cid * NS * CH0 + sid * jnp.where(cid == 0, CH0, CH1)) * CHUNK

    def body(g, carry):
        @pl.when(g < my_ch)
        def _():
            base = base0 + g * CHUNK
            pltpu.sync_copy(ej_hbm.at[pl.ds(base, CHUNK)], idxj_v)
            pltpu.async_copy(m_hbm.at[idxj_v], rows_v, gsem).wait()
            pltpu.sync_copy(ei_hbm.at[pl.ds(base, CHUNK)], idxi_v)
            pltpu.sync_copy(rows_v, acc_sh.at[idxi_v], add=True)
        return carry

    lax.fori_loop(0, max(CH0, CH1), body, 0)
    plsc.subcore_barrier()
    pltpu.sync_copy(acc_sh.at[pl.ds(sid * rpt, rpt)],
                    out_hbm.at[cid, pl.ds(sid * rpt, rpt)])


@functools.cache
def _make_segsum():
    return functools.partial(
        pl.kernel,
        mesh=plsc.VectorSubcoreMesh(core_axis_name="c", subcore_axis_name="s"),
        out_type=jax.ShapeDtypeStruct((NC, ACC_ROWS, EMB), jnp.float32),
        scratch_types=[
            pltpu.VMEM((CHUNK,), jnp.int32),
            pltpu.VMEM((CHUNK,), jnp.int32),
            pltpu.VMEM((CHUNK, EMB), jnp.float32),
            pltpu.VMEM_SHARED((ACC_ROWS, EMB), jnp.float32),
            pltpu.SemaphoreType.DMA,
        ],
    )(_segsum_body)


def _segsum(m, ej, ei, zeros_src):
    return _make_segsum()(m, ej, ei, zeros_src)


def _row_specs(n):
    return [pl.BlockSpec((ABLK, EMB), lambda i: (i, 0)) for _ in range(n)]


def _full(shape):
    return pl.BlockSpec(shape, lambda i: tuple(0 for _ in shape))


def kernel(Z, N, R, edge_id_i, edge_id_j, coords, emb_table, Wm, bm, Wu, bu,
           Wo1, bo1, Ws, bs, Wp, bp, alpha_s, alpha_p):
    A = Z.shape[0]
    E = edge_id_i.shape[0]
    G = coords.shape[0]
    grid = (A // ABLK,)

    emb_pad = jnp.zeros((EMB, EMB), jnp.float32).at[:emb_table.shape[0]].set(emb_table)
    e_pad = NS * (CH0 + CH1) * CHUNK
    assert e_pad >= E
    ej = edge_id_j.astype(jnp.int32)
    ei = edge_id_i.astype(jnp.int32)
    if e_pad > E:
        ej = jnp.concatenate([ej, jnp.zeros((e_pad - E,), jnp.int32)])
        ei = jnp.concatenate([ei, jnp.full((e_pad - E,), A, jnp.int32)])
    zeros_src = jnp.zeros((ACC_ROWS // NS, EMB), jnp.float32)

    h, m = pl.pallas_call(
        _embed_body,
        grid=grid,
        in_specs=[pl.BlockSpec((ABLK, 1), lambda i: (i, 0)),
                  _full((EMB, EMB)), _full((EMB, EMB)), _full((1, EMB))],
        out_specs=_row_specs(2),
        out_shape=[jax.ShapeDtypeStruct((A, EMB), jnp.float32)] * 2,
    )(Z.reshape(A, 1).astype(jnp.int32), emb_pad, Wm[0], bm[0].reshape(1, EMB))

    NUM_BLOCKS = Wm.shape[0]
    for t in range(NUM_BLOCKS):
        parts = _segsum(m, ej, ei, zeros_src)
        if t < NUM_BLOCKS - 1:
            wn, bn = Wm[t + 1], bm[t + 1]
        else:
            wn, bn = Wo1, bo1
        h, m = pl.pallas_call(
            _update_body,
            grid=grid,
            in_specs=[pl.BlockSpec((NC, ABLK, EMB), lambda i: (0, i, 0)),
                      pl.BlockSpec((ABLK, EMB), lambda i: (i, 0)),
                      _full((EMB, EMB)), _full((1, EMB)),
                      _full((EMB, EMB)), _full((1, EMB))],
            out_specs=_row_specs(2),
            out_shape=[jax.ShapeDtypeStruct((A, EMB), jnp.float32)] * 2,
        )(parts, h, Wu[t], bu[t].reshape(1, EMB), wn, bn.reshape(1, EMB))

    wsp = jnp.zeros((EMB, 16), jnp.float32).at[:, 0:4].set(Ws).at[:, 4:10].set(Wp)
    bsp = jnp.zeros((1, 16), jnp.float32).at[0, 0:4].set(bs).at[0, 4:10].set(bp)
    rpad = jnp.zeros((A, 8), jnp.float32).at[:, 0:3].set(R)
    ct = jnp.zeros((8, G), jnp.float32).at[0:3, :].set(coords.T)
    al = jnp.zeros((1, 8), jnp.float32).at[0, 0:4].set(alpha_s).at[0, 4:6].set(alpha_p)
    dens = pl.pallas_call(
        _dens_body,
        grid=grid,
        in_specs=[pl.BlockSpec((ABLK, EMB), lambda i: (i, 0)),
                  _full((EMB, 16)), _full((1, 16)),
                  pl.BlockSpec((ABLK, 8), lambda i: (i, 0)),
                  _full((8, G)), _full((1, 8))],
        out_specs=_full((1, G)),
        out_shape=jax.ShapeDtypeStruct((1, G), jnp.float32),
    )(m, wsp, bsp, rpad, ct, al)
    return dens.reshape(G)


# speedup vs baseline: 1.8001x; 1.0046x over previous
import functools

import jax
import jax.numpy as jnp
from jax import lax
from jax.experimental import pallas as pl
from jax.experimental.pallas import tpu as pltpu
from jax.experimental.pallas import tpu_sc as plsc

EMB = 128
ABLK = 2000
NC = 2
NS = 16
CHUNK = 128
ACC_ROWS = 10112


def _swish(x):
    return x * jax.nn.sigmoid(x)


def _embed_body(z_ref, emb_ref, wm_ref, bm_ref, h_ref, m_ref):
    z = z_ref[...]
    io = lax.broadcasted_iota(jnp.int32, (z.shape[0], EMB), 1)
    oh = (io == z).astype(jnp.float32)
    h = jnp.dot(oh, emb_ref[...], preferred_element_type=jnp.float32, precision=jax.lax.Precision.HIGHEST)
    h_ref[...] = h
    m_ref[...] = _swish(
        jnp.dot(h, wm_ref[...], preferred_element_type=jnp.float32, precision=jax.lax.Precision.HIGHEST) + bm_ref[...])


def _update_body(p_ref, h_ref, wu_ref, bu_ref, wn_ref, bn_ref, hn_ref, mn_ref):
    agg = p_ref[0] + p_ref[1]
    u = _swish(
        jnp.dot(agg, wu_ref[...], preferred_element_type=jnp.float32, precision=jax.lax.Precision.HIGHEST) + bu_ref[...])
    hn = h_ref[...] + u
    hn_ref[...] = hn
    mn_ref[...] = _swish(
        jnp.dot(hn, wn_ref[...], preferred_element_type=jnp.float32, precision=jax.lax.Precision.HIGHEST) + bn_ref[...])


def _dens_body(ho_ref, wsp_ref, bsp_ref, r_ref, ct_ref, al_ref, o_ref):
    sp = jnp.dot(ho_ref[...], wsp_ref[...],
                 preferred_element_type=jnp.float32, precision=jax.lax.Precision.HIGHEST) + bsp_ref[...]
    rb = r_ref[...]
    ct = ct_ref[...]
    a = al_ref[...]
    d0 = rb[:, 0:1] - ct[0:1, :]
    d1 = rb[:, 1:2] - ct[1:2, :]
    d2 = rb[:, 2:3] - ct[2:3, :]
    r2 = d0 * d0 + d1 * d1 + d2 * d2
    acc = jnp.zeros_like(r2)
    for s in range(4):
        acc = acc + sp[:, s:s + 1] * jnp.exp(r2 * (-jnp.abs(a[0:1, s:s + 1])))
    for p in range(2):
        cpp = sp[:, 4 + 3 * p:7 + 3 * p]
        pdc = jnp.dot(cpp, ct[0:3, :], preferred_element_type=jnp.float32, precision=jax.lax.Precision.HIGHEST)
        prd = jnp.sum(cpp * rb[:, 0:3], axis=1, keepdims=True)
        acc = acc + (pdc - prd) * jnp.exp(r2 * (-jnp.abs(a[0:1, 4 + p:5 + p])))
    col = jnp.sum(acc, axis=0, keepdims=True)

    @pl.when(pl.program_id(0) == 0)
    def _():
        o_ref[...] = jnp.zeros_like(o_ref)

    o_ref[...] += col


CH0 = 79
CH1 = 78


def _segsum_body(m_hbm, ej_hbm, ei_hbm, z_hbm, out_hbm,
                 idxj_v, idxi_v, rows_v, acc_sh, gsem):
    cid = lax.axis_index("c")
    sid = lax.axis_index("s")
    rpt = ACC_ROWS // NS
    pltpu.sync_copy(z_hbm, acc_sh.at[pl.ds(sid * rpt, rpt)])
    plsc.subcore_barrier()
    my_ch = jnp.where(cid == 0, CH0, CH1)
    base0 = (cid * NS * CH0 + sid * jnp.where(cid == 0, CH0, CH1)) * CHUNK

    def body(g, carry):
        @pl.when(g < my_ch)
        def _():
            base = base0 + g * CHUNK
            pltpu.sync_copy(ej_hbm.at[pl.ds(base, CHUNK)], idxj_v)
            pltpu.async_copy(m_hbm.at[idxj_v], rows_v, gsem).wait()
            pltpu.sync_copy(ei_hbm.at[pl.ds(base, CHUNK)], idxi_v)
            pltpu.sync_copy(rows_v, acc_sh.at[idxi_v], add=True)
        return carry

    lax.fori_loop(0, max(CH0, CH1), body, 0)
    plsc.subcore_barrier()
    pltpu.sync_copy(acc_sh.at[pl.ds(sid * rpt, rpt)],
                    out_hbm.at[cid, pl.ds(sid * rpt, rpt)])


@functools.cache
def _make_segsum():
    return functools.partial(
        pl.kernel,
        mesh=plsc.VectorSubcoreMesh(core_axis_name="c", subcore_axis_name="s"),
        out_type=jax.ShapeDtypeStruct((NC, ACC_ROWS, EMB), jnp.float32),
        scratch_types=[
            pltpu.VMEM((CHUNK,), jnp.int32),
            pltpu.VMEM((CHUNK,), jnp.int32),
            pltpu.VMEM((CHUNK, EMB), jnp.float32),
            pltpu.VMEM_SHARED((ACC_ROWS, EMB), jnp.float32),
            pltpu.SemaphoreType.DMA,
        ],
    )(_segsum_body)


def _segsum(m, ej, ei, zeros_src):
    return _make_segsum()(m, ej, ei, zeros_src)


def _row_specs(n):
    return [pl.BlockSpec((ABLK, EMB), lambda i: (i, 0)) for _ in range(n)]


def _full(shape):
    return pl.BlockSpec(shape, lambda i: tuple(0 for _ in shape))


def kernel(Z, N, R, edge_id_i, edge_id_j, coords, emb_table, Wm, bm, Wu, bu,
           Wo1, bo1, Ws, bs, Wp, bp, alpha_s, alpha_p):
    A = Z.shape[0]
    E = edge_id_i.shape[0]
    G = coords.shape[0]
    grid = (A // ABLK,)

    emb_pad = jnp.zeros((EMB, EMB), jnp.float32).at[:emb_table.shape[0]].set(emb_table)
    e_pad = NS * (CH0 + CH1) * CHUNK
    assert e_pad >= E
    ej = edge_id_j.astype(jnp.int32)
    ei = edge_id_i.astype(jnp.int32)
    if e_pad > E:
        ej = jnp.concatenate([ej, jnp.zeros((e_pad - E,), jnp.int32)])
        ei = jnp.concatenate([ei, jnp.full((e_pad - E,), A, jnp.int32)])
    zeros_src = jnp.zeros((ACC_ROWS // NS, EMB), jnp.float32)

    h, m = pl.pallas_call(
        _embed_body,
        grid=grid,
        in_specs=[pl.BlockSpec((ABLK, 1), lambda i: (i, 0)),
                  _full((EMB, EMB)), _full((EMB, EMB)), _full((1, EMB))],
        out_specs=_row_specs(2),
        out_shape=[jax.ShapeDtypeStruct((A, EMB), jnp.float32)] * 2,
    )(Z.reshape(A, 1).astype(jnp.int32), emb_pad, Wm[0], bm[0].reshape(1, EMB))

    NUM_BLOCKS = Wm.shape[0]
    for t in range(NUM_BLOCKS):
        parts = _segsum(m, ej, ei, zeros_src)
        if t < NUM_BLOCKS - 1:
            wn, bn = Wm[t + 1], bm[t + 1]
        else:
            wn, bn = Wo1, bo1
        h, m = pl.pallas_call(
            _update_body,
            grid=grid,
            in_specs=[pl.BlockSpec((NC, ABLK, EMB), lambda i: (0, i, 0)),
                      pl.BlockSpec((ABLK, EMB), lambda i: (i, 0)),
                      _full((EMB, EMB)), _full((1, EMB)),
                      _full((EMB, EMB)), _full((1, EMB))],
            out_specs=_row_specs(2),
            out_shape=[jax.ShapeDtypeStruct((A, EMB), jnp.float32)] * 2,
        )(parts, h, Wu[t], bu[t].reshape(1, EMB), wn, bn.reshape(1, EMB))

    wsp = jnp.zeros((EMB, 16), jnp.float32).at[:, 0:4].set(Ws).at[:, 4:10].set(Wp)
    bsp = jnp.zeros((1, 16), jnp.float32).at[0, 0:4].set(bs).at[0, 4:10].set(bp)
    rpad = jnp.zeros((A, 8), jnp.float32).at[:, 0:3].set(R)
    ct = jnp.zeros((8, G), jnp.float32).at[0:3, :].set(coords.T)
    al = jnp.zeros((1, 8), jnp.float32).at[0, 0:4].set(alpha_s).at[0, 4:6].set(alpha_p)
    dens = pl.pallas_call(
        _dens_body,
        grid=grid,
        in_specs=[pl.BlockSpec((ABLK, EMB), lambda i: (i, 0)),
                  _full((EMB, 16)), _full((1, 16)),
                  pl.BlockSpec((ABLK, 8), lambda i: (i, 0)),
                  _full((8, G)), _full((1, 8))],
        out_specs=_full((1, G)),
        out_shape=jax.ShapeDtypeStruct((1, G), jnp.float32),
    )(m, wsp, bsp, rpad, ct, al)
    return dens.reshape(G)
